# Initial kernel scaffold; baseline (speedup 1.0000x reference)
import jax
import jax.numpy as jnp
from jax.experimental import pallas as pl


def kernel(feat, edge_index, node_graph_ids, W1, b1, W2, b2, eps, gn_weight, gn_bias, gn_scale, pW1, pb1, pW2, pb2):
    raise NotImplementedError("write your pallas kernel here")


# speedup vs baseline: 4.3062x; 4.3062x over previous
import functools

import jax
import jax.numpy as jnp
from jax import lax
from jax.experimental import pallas as pl
from jax.experimental.pallas import tpu as pltpu
from jax.experimental.pallas import tpu_sc as plsc

G = 64
NC = 2
NS = 16
NW = NC * NS
CHUNK = 128


def _sc_aggregate(n_acc: int, cpt: int, d: int):
    rows_per_tile = n_acc // NS
    mesh = plsc.VectorSubcoreMesh(
        core_axis_name="c", subcore_axis_name="s", num_cores=NC, num_subcores=NS
    )

    @functools.partial(
        pl.kernel,
        mesh=mesh,
        out_type=jax.ShapeDtypeStruct((NC, n_acc, d), jnp.float32),
        scratch_types=[
            pltpu.VMEM((cpt, CHUNK), jnp.int32),
            pltpu.VMEM((cpt, CHUNK), jnp.int32),
            pltpu.VMEM((CHUNK, d), jnp.float32),
            pltpu.VMEM_SHARED((n_acc, d), jnp.float32),
            pltpu.SemaphoreType.DMA,
        ],
    )
    def agg(x_hbm, src_hbm, dst_hbm, zero_hbm, out_hbm, src_v, dst_v, rows_v, acc_sh, sem):
        c = lax.axis_index("c")
        s = lax.axis_index("s")
        wid = c * NS + s
        base = s * rows_per_tile
        pltpu.sync_copy(zero_hbm.at[pl.ds(base, rows_per_tile)],
                        acc_sh.at[pl.ds(base, rows_per_tile)])
        pltpu.sync_copy(src_hbm.at[wid], src_v)
        pltpu.sync_copy(dst_hbm.at[wid], dst_v)
        plsc.subcore_barrier()

        def body(j, carry):
            pltpu.async_copy(x_hbm.at[src_v.at[j]], rows_v, sem).wait()
            pltpu.sync_copy(rows_v, acc_sh.at[dst_v.at[j]], add=True)
            return carry

        lax.fori_loop(0, cpt, body, 0)
        plsc.subcore_barrier()
        pltpu.sync_copy(acc_sh.at[pl.ds(base, rows_per_tile)],
                        out_hbm.at[c, pl.ds(base, rows_per_tile)])

    return agg


def _dot(a, b):
    return lax.dot_general(a, b, (((1,), (0,)), ((), ())),
                           preferred_element_type=jnp.float32,
                           precision=lax.Precision.HIGHEST)


def _dott(a, b):
    return lax.dot_general(a, b, (((0,), (0,)), ((), ())),
                           preferred_element_type=jnp.float32,
                           precision=lax.Precision.HIGHEST)


def _onehot(gid_ref, n):
    return (lax.broadcasted_iota(jnp.int32, (n, G), 1) == gid_ref[...]).astype(jnp.float32)


def _tc_mlp_body(x_ref, a0_ref, a1_ref, gid_ref, eps_ref,
                 w1_ref, b1_ref, w2_ref, b2_ref,
                 h_out_ref, s1_out_ref, cnt_out_ref):
    n = x_ref.shape[0]
    h0 = x_ref[...] * eps_ref[...] + (a0_ref[...] + a1_ref[...])
    t = jnp.maximum(_dot(h0, w1_ref[...]) + b1_ref[...], 0.0)
    h = _dot(t, w2_ref[...]) + b2_ref[...]
    oh = _onehot(gid_ref, n)
    cnt = jnp.maximum(_dott(oh, jnp.ones((n, 1), jnp.float32)), 1.0)
    h_out_ref[...] = h
    s1_out_ref[...] = _dott(oh, h) / cnt
    cnt_out_ref[...] = cnt


def _tc_norm_body(h_ref, gid_ref, s1_ref, cnt_ref,
                  gnw_ref, gnb_ref, gns_ref,
                  pw1_ref, pb1_ref, pw2_ref, pb2_ref,
                  h_out_ref, pool_out_ref):
    n = h_ref.shape[0]
    oh = _onehot(gid_ref, n)
    sub = h_ref[...] - _dot(oh, s1_ref[...]) * gns_ref[...]
    s2 = _dott(oh, sub * sub) / cnt_ref[...]
    var_b = _dot(oh, s2)
    h2 = jnp.maximum(gnw_ref[...] * sub * lax.rsqrt(var_b + 1e-8) + gnb_ref[...], 0.0)
    u = jnp.maximum(_dot(h2, pw1_ref[...]) + pb1_ref[...], 0.0)
    z = _dot(u, pw2_ref[...]) + pb2_ref[...]
    h_out_ref[...] = h2
    pool_out_ref[...] = _dott(oh, z)


def kernel(feat, edge_index, node_graph_ids, W1, b1, W2, b2, eps,
           gn_weight, gn_bias, gn_scale, pW1, pb1, pW2, pb2):
    n, d = feat.shape
    e = edge_index.shape[1]
    t_out = pW2.shape[2]
    num_layers = W1.shape[0]

    cpt = -(-e // (NW * CHUNK))
    e_pad = NW * cpt * CHUNK
    n_acc = -(-(n + 1) // (NS * 8)) * (NS * 8)

    src = jnp.concatenate(
        [edge_index[0], jnp.zeros((e_pad - e,), jnp.int32)]).reshape(NW, cpt, CHUNK)
    dst = jnp.concatenate(
        [edge_index[1], jnp.full((e_pad - e,), n, jnp.int32)]).reshape(NW, cpt, CHUNK)
    zeros_blk = jnp.zeros((n_acc, d), jnp.float32)
    gid2 = node_graph_ids.reshape(n, 1)

    sc_agg = _sc_aggregate(n_acc, cpt, d)

    tc_mlp = pl.pallas_call(
        _tc_mlp_body,
        out_shape=(jax.ShapeDtypeStruct((n, d), jnp.float32),
                   jax.ShapeDtypeStruct((G, d), jnp.float32),
                   jax.ShapeDtypeStruct((G, 1), jnp.float32)),
    )
    tc_norm = pl.pallas_call(
        _tc_norm_body,
        out_shape=(jax.ShapeDtypeStruct((n, d), jnp.float32),
                   jax.ShapeDtypeStruct((G, t_out), jnp.float32)),
    )

    x = feat
    pooled = []
    for l in range(num_layers):
        x_pad = jnp.concatenate([x, jnp.zeros((n_acc - n, d), jnp.float32)])
        part = sc_agg(x_pad, src, dst, zeros_blk)
        epsv = jnp.full((1, d), 1.0 + eps[l], jnp.float32)
        h, s1, cnt = tc_mlp(
            x, part[0, :n], part[1, :n], gid2, epsv,
            W1[l], b1[l].reshape(1, d), W2[l], b2[l].reshape(1, d))
        x, pool = tc_norm(
            h, gid2, s1, cnt,
            gn_weight[l].reshape(1, d), gn_bias[l].reshape(1, d),
            gn_scale[l].reshape(1, d),
            pW1[l], pb1[l].reshape(1, d), pW2[l], pb2[l].reshape(1, t_out))
        pooled.append(pool)
    return jnp.concatenate(pooled, axis=-1)


# speedup vs baseline: 4.8042x; 1.1156x over previous
import functools

import jax
import jax.numpy as jnp
from jax import lax
from jax.experimental import pallas as pl
from jax.experimental.pallas import tpu as pltpu
from jax.experimental.pallas import tpu_sc as plsc

G = 64
NC = 2
NS = 16
NW = NC * NS
CHUNK = 128


def _sc_aggregate(n_acc: int, cpt: int, d: int):
    rows_per_tile = n_acc // NS
    mesh = plsc.VectorSubcoreMesh(
        core_axis_name="c", subcore_axis_name="s", num_cores=NC, num_subcores=NS
    )

    @functools.partial(
        pl.kernel,
        mesh=mesh,
        out_type=jax.ShapeDtypeStruct((NC, n_acc, d), jnp.float32),
        scratch_types=[
            pltpu.VMEM((cpt, CHUNK), jnp.int32),
            pltpu.VMEM((2, CHUNK), jnp.int32),
            pltpu.VMEM((2, CHUNK, d), jnp.float32),
            pltpu.VMEM_SHARED((n_acc, d), jnp.float32),
            pltpu.SemaphoreType.DMA,
            pltpu.SemaphoreType.DMA,
        ],
    )
    def agg(x_hbm, src_hbm, dst_hbm, zero_hbm, out_hbm, src_v, dst_v, rows_v, acc_sh, gsem, ssem):
        c = lax.axis_index("c")
        s = lax.axis_index("s")
        wid = c * NS + s
        base = s * rows_per_tile
        pltpu.sync_copy(zero_hbm.at[pl.ds(base, rows_per_tile)],
                        acc_sh.at[pl.ds(base, rows_per_tile)])
        pltpu.sync_copy(src_hbm.at[wid], src_v)
        plsc.subcore_barrier()

        pltpu.async_copy(x_hbm.at[src_v.at[0]], rows_v.at[0], gsem)
        pltpu.async_copy(dst_hbm.at[wid, 0], dst_v.at[0], gsem)

        def body(j, carry):
            p = lax.rem(j, 2)
            pltpu.make_async_copy(x_hbm.at[pl.ds(0, CHUNK)], rows_v.at[p], gsem).wait()
            pltpu.make_async_copy(dst_hbm.at[0, 0], dst_v.at[p], gsem).wait()

            @pl.when(j >= 1)
            def _():
                pltpu.make_async_copy(rows_v.at[1 - p],
                                      acc_sh.at[pl.ds(0, CHUNK)], ssem).wait()

            @pl.when(j + 1 < cpt)
            def _():
                pltpu.async_copy(x_hbm.at[src_v.at[j + 1]], rows_v.at[1 - p], gsem)
                pltpu.async_copy(dst_hbm.at[wid, j + 1], dst_v.at[1 - p], gsem)

            pltpu.async_copy(rows_v.at[p], acc_sh.at[dst_v.at[p]], ssem, add=True)
            return carry

        lax.fori_loop(0, cpt, body, 0)
        pltpu.make_async_copy(rows_v.at[lax.rem(cpt - 1, 2)],
                              acc_sh.at[pl.ds(0, CHUNK)], ssem).wait()
        plsc.subcore_barrier()
        pltpu.sync_copy(acc_sh.at[pl.ds(base, rows_per_tile)],
                        out_hbm.at[c, pl.ds(base, rows_per_tile)])

    return agg


def _dot(a, b):
    return lax.dot_general(a, b, (((1,), (0,)), ((), ())),
                           preferred_element_type=jnp.float32,
                           precision=lax.Precision.HIGHEST)


def _dott(a, b):
    return lax.dot_general(a, b, (((0,), (0,)), ((), ())),
                           preferred_element_type=jnp.float32,
                           precision=lax.Precision.HIGHEST)


def _onehot(gid_ref, n):
    return (lax.broadcasted_iota(jnp.int32, (n, G), 1) == gid_ref[...]).astype(jnp.float32)


def _tc_mlp_body(x_ref, a0_ref, a1_ref, gid_ref, eps_ref,
                 w1_ref, b1_ref, w2_ref, b2_ref,
                 h_out_ref, s1_out_ref, cnt_out_ref):
    n = x_ref.shape[0]
    h0 = x_ref[...] * eps_ref[...] + (a0_ref[...] + a1_ref[...])
    t = jnp.maximum(_dot(h0, w1_ref[...]) + b1_ref[...], 0.0)
    h = _dot(t, w2_ref[...]) + b2_ref[...]
    oh = _onehot(gid_ref, n)
    cnt = jnp.maximum(_dott(oh, jnp.ones((n, 1), jnp.float32)), 1.0)
    h_out_ref[...] = h
    s1_out_ref[...] = _dott(oh, h) / cnt
    cnt_out_ref[...] = cnt


def _tc_norm_body(h_ref, gid_ref, s1_ref, cnt_ref,
                  gnw_ref, gnb_ref, gns_ref,
                  pw1_ref, pb1_ref, pw2_ref, pb2_ref,
                  h_out_ref, pool_out_ref):
    n = h_ref.shape[0]
    oh = _onehot(gid_ref, n)
    sub = h_ref[...] - _dot(oh, s1_ref[...]) * gns_ref[...]
    s2 = _dott(oh, sub * sub) / cnt_ref[...]
    var_b = _dot(oh, s2)
    h2 = jnp.maximum(gnw_ref[...] * sub * lax.rsqrt(var_b + 1e-8) + gnb_ref[...], 0.0)
    u = jnp.maximum(_dot(h2, pw1_ref[...]) + pb1_ref[...], 0.0)
    z = _dot(u, pw2_ref[...]) + pb2_ref[...]
    h_out_ref[...] = h2
    pool_out_ref[...] = _dott(oh, z)


def kernel(feat, edge_index, node_graph_ids, W1, b1, W2, b2, eps,
           gn_weight, gn_bias, gn_scale, pW1, pb1, pW2, pb2):
    n, d = feat.shape
    e = edge_index.shape[1]
    t_out = pW2.shape[2]
    num_layers = W1.shape[0]

    cpt = -(-e // (NW * CHUNK))
    e_pad = NW * cpt * CHUNK
    n_acc = -(-(n + 1) // (NS * 8)) * (NS * 8)

    src = jnp.concatenate(
        [edge_index[0], jnp.zeros((e_pad - e,), jnp.int32)]).reshape(NW, cpt, CHUNK)
    dst = jnp.concatenate(
        [edge_index[1], jnp.full((e_pad - e,), n, jnp.int32)]).reshape(NW, cpt, CHUNK)
    zeros_blk = jnp.zeros((n_acc, d), jnp.float32)
    gid2 = node_graph_ids.reshape(n, 1)

    sc_agg = _sc_aggregate(n_acc, cpt, d)

    tc_mlp = pl.pallas_call(
        _tc_mlp_body,
        out_shape=(jax.ShapeDtypeStruct((n, d), jnp.float32),
                   jax.ShapeDtypeStruct((G, d), jnp.float32),
                   jax.ShapeDtypeStruct((G, 1), jnp.float32)),
    )
    tc_norm = pl.pallas_call(
        _tc_norm_body,
        out_shape=(jax.ShapeDtypeStruct((n, d), jnp.float32),
                   jax.ShapeDtypeStruct((G, t_out), jnp.float32)),
    )

    x = feat
    pooled = []
    for l in range(num_layers):
        x_pad = jnp.concatenate([x, jnp.zeros((n_acc - n, d), jnp.float32)])
        part = sc_agg(x_pad, src, dst, zeros_blk)
        epsv = jnp.full((1, d), 1.0 + eps[l], jnp.float32)
        h, s1, cnt = tc_mlp(
            x, part[0, :n], part[1, :n], gid2, epsv,
            W1[l], b1[l].reshape(1, d), W2[l], b2[l].reshape(1, d))
        x, pool = tc_norm(
            h, gid2, s1, cnt,
            gn_weight[l].reshape(1, d), gn_bias[l].reshape(1, d),
            gn_scale[l].reshape(1, d),
            pW1[l], pb1[l].reshape(1, d), pW2[l], pb2[l].reshape(1, t_out))
        pooled.append(pool)
    return jnp.concatenate(pooled, axis=-1)
